# baseline (speedup 1.0000x reference)
import jax
import jax.numpy as jnp
from jax.experimental import pallas as pl
from jax.experimental.pallas import tpu as pltpu

_TEMP_INV = 10.0
_N = 2048
_R = 8
_BI = 256
_H = 512
_NH = _N // _H
_NS = _H // _BI


def _body(pr_ref, tr_ref, pc_ref, tc_ref, out_ref, acc_ref):
    r = pl.program_id(0)
    h = pl.program_id(1)

    @pl.when(jnp.logical_and(r == 0, h == 0))
    def _():
        acc_ref[3] = 0.0

    @pl.when(h == 0)
    def _():
        acc_ref[0] = 0.0
        acc_ref[1] = 0.0
        acc_ref[2] = 0.0

    scale = 0.5 * _TEMP_INV
    pcb = jnp.broadcast_to((pc_ref[0, 0, :] * scale).reshape(_N, 1), (_N, _BI))
    tcb = jnp.broadcast_to((tc_ref[0, 0, :] * scale).reshape(_N, 1), (_N, _BI))

    ones_row = jnp.ones((1, _N), jnp.bfloat16)
    dims = (((1,), (0,)), ((), ()))

    sxy = 0.0
    sxx = 0.0
    syy = 0.0
    for s in range(_NS):
        pi = (pr_ref[0, 0, s * _BI:(s + 1) * _BI] * scale).reshape(1, _BI)
        ti = (tr_ref[0, 0, s * _BI:(s + 1) * _BI] * scale).reshape(1, _BI)
        bp = jnp.tanh((pi - pcb).astype(jnp.bfloat16))
        bt = jnp.tanh((ti - tcb).astype(jnp.bfloat16))
        xb = 0.5 * jax.lax.dot_general(ones_row, bp, dims,
                                       preferred_element_type=jnp.float32)
        yb = 0.5 * jax.lax.dot_general(ones_row, bt, dims,
                                       preferred_element_type=jnp.float32)
        sxy += jnp.sum(xb * yb)
        sxx += jnp.sum(xb * xb)
        syy += jnp.sum(yb * yb)

    acc_ref[0] += sxy
    acc_ref[1] += sxx
    acc_ref[2] += syy

    @pl.when(h == _NH - 1)
    def _():
        vx = jnp.sqrt(acc_ref[1] / _N + 1e-8)
        vy = jnp.sqrt(acc_ref[2] / _N + 1e-8)
        rho = (acc_ref[0] / _N) / (vx * vy + 1e-8)
        acc_ref[3] += (1.0 - rho) / _R

    @pl.when(jnp.logical_and(r == _R - 1, h == _NH - 1))
    def _():
        out_ref[0, 0] = acc_ref[3]


def kernel(preds, targets):
    ap_row = preds.reshape(_R, 1, _N)
    at_row = targets.reshape(_R, 1, _N)
    out = pl.pallas_call(
        _body,
        grid=(_R, _NH),
        in_specs=[
            pl.BlockSpec((1, 1, _H), lambda r, h: (r, 0, h)),
            pl.BlockSpec((1, 1, _H), lambda r, h: (r, 0, h)),
            pl.BlockSpec((1, 1, _N), lambda r, h: (r, 0, 0)),
            pl.BlockSpec((1, 1, _N), lambda r, h: (r, 0, 0)),
        ],
        out_specs=pl.BlockSpec(memory_space=pltpu.SMEM),
        out_shape=jax.ShapeDtypeStruct((1, 1), jnp.float32),
        scratch_shapes=[pltpu.SMEM((4,), jnp.float32)],
    )(ap_row, at_row, ap_row, at_row)
    return out[0, 0]


# speedup vs baseline: 1.0553x; 1.0553x over previous
import jax
import jax.numpy as jnp
from jax.experimental import pallas as pl
from jax.experimental.pallas import tpu as pltpu

_TEMP_INV = 10.0
_N = 2048
_R = 8
_BI = 256
_H = 1024
_NH = _N // _H
_NS = _H // _BI


def _body(pr_ref, tr_ref, pc_ref, tc_ref, out_ref, acc_ref):
    r = pl.program_id(0)
    h = pl.program_id(1)

    @pl.when(jnp.logical_and(r == 0, h == 0))
    def _():
        acc_ref[3] = 0.0

    @pl.when(h == 0)
    def _():
        acc_ref[0] = 0.0
        acc_ref[1] = 0.0
        acc_ref[2] = 0.0

    scale = 0.5 * _TEMP_INV
    pcb = jnp.broadcast_to((pc_ref[0, 0, :] * scale).reshape(_N, 1), (_N, _BI))
    tcb = jnp.broadcast_to((tc_ref[0, 0, :] * scale).reshape(_N, 1), (_N, _BI))

    ones_row = jnp.ones((1, _N), jnp.bfloat16)
    dims = (((1,), (0,)), ((), ()))

    sxy = 0.0
    sxx = 0.0
    syy = 0.0
    for s in range(_NS):
        pi = (pr_ref[0, 0, s * _BI:(s + 1) * _BI] * scale).reshape(1, _BI)
        ti = (tr_ref[0, 0, s * _BI:(s + 1) * _BI] * scale).reshape(1, _BI)
        bp = jnp.tanh((pi - pcb).astype(jnp.bfloat16))
        bt = jnp.tanh((ti - tcb).astype(jnp.bfloat16))
        xb = 0.5 * jax.lax.dot_general(ones_row, bp, dims,
                                       preferred_element_type=jnp.float32)
        yb = 0.5 * jax.lax.dot_general(ones_row, bt, dims,
                                       preferred_element_type=jnp.float32)
        sxy += jnp.sum(xb * yb)
        sxx += jnp.sum(xb * xb)
        syy += jnp.sum(yb * yb)

    acc_ref[0] += sxy
    acc_ref[1] += sxx
    acc_ref[2] += syy

    @pl.when(h == _NH - 1)
    def _():
        vx = jnp.sqrt(acc_ref[1] / _N + 1e-8)
        vy = jnp.sqrt(acc_ref[2] / _N + 1e-8)
        rho = (acc_ref[0] / _N) / (vx * vy + 1e-8)
        acc_ref[3] += (1.0 - rho) / _R

    @pl.when(jnp.logical_and(r == _R - 1, h == _NH - 1))
    def _():
        out_ref[0, 0] = acc_ref[3]


def kernel(preds, targets):
    ap_row = preds.reshape(_R, 1, _N)
    at_row = targets.reshape(_R, 1, _N)
    out = pl.pallas_call(
        _body,
        grid=(_R, _NH),
        in_specs=[
            pl.BlockSpec((1, 1, _H), lambda r, h: (r, 0, h)),
            pl.BlockSpec((1, 1, _H), lambda r, h: (r, 0, h)),
            pl.BlockSpec((1, 1, _N), lambda r, h: (r, 0, 0)),
            pl.BlockSpec((1, 1, _N), lambda r, h: (r, 0, 0)),
        ],
        out_specs=pl.BlockSpec(memory_space=pltpu.SMEM),
        out_shape=jax.ShapeDtypeStruct((1, 1), jnp.float32),
        scratch_shapes=[pltpu.SMEM((4,), jnp.float32)],
    )(ap_row, at_row, ap_row, at_row)
    return out[0, 0]


# speedup vs baseline: 1.0635x; 1.0078x over previous
import jax
import jax.numpy as jnp
from jax.experimental import pallas as pl
from jax.experimental.pallas import tpu as pltpu

_TEMP_INV = 10.0
_N = 2048
_R = 8
_BI = 256
_NK = _N // _BI


def _body(p_ref, t_ref, out_ref, acc_ref, tp_ref, tt_ref):
    r = pl.program_id(0)

    @pl.when(r == 0)
    def _():
        acc_ref[0] = 0.0

    tp_ref[0, :] = jnp.zeros((_N,), jnp.float32)
    tt_ref[0, :] = jnp.zeros((_N,), jnp.float32)

    ap = p_ref[0, 0, :] * (0.5 * _TEMP_INV)
    at = t_ref[0, 0, :] * (0.5 * _TEMP_INV)

    sxy = 0.0
    sxx = 0.0
    syy = 0.0
    for i in range(_NK):
        lo = i * _BI
        hi = (i + 1) * _BI
        w = _N - lo

        pi = ap[lo:hi].reshape(_BI, 1)
        ps = ap[lo:].reshape(1, w)
        ti = at[lo:hi].reshape(_BI, 1)
        ts = at[lo:].reshape(1, w)

        bp_h = jnp.tanh((pi - ps).astype(jnp.bfloat16))
        bt_h = jnp.tanh((ti - ts).astype(jnp.bfloat16))
        ones_col = jnp.ones((w, 1), jnp.bfloat16)
        ones_row = jnp.ones((1, _BI), jnp.bfloat16)
        dims = (((1,), (0,)), ((), ()))
        rs_p = jax.lax.dot_general(bp_h, ones_col, dims,
                                   preferred_element_type=jnp.float32)
        rs_t = jax.lax.dot_general(bt_h, ones_col, dims,
                                   preferred_element_type=jnp.float32)
        cs_p = jax.lax.dot_general(ones_row, bp_h, dims,
                                   preferred_element_type=jnp.float32)
        cs_t = jax.lax.dot_general(ones_row, bt_h, dims,
                                   preferred_element_type=jnp.float32)

        xb = 0.5 * (tp_ref[0, lo:hi] + rs_p.reshape(_BI))
        yb = 0.5 * (tt_ref[0, lo:hi] + rs_t.reshape(_BI))
        if i < _NK - 1:
            tp_ref[0, hi:] -= cs_p[0, _BI:]
            tt_ref[0, hi:] -= cs_t[0, _BI:]

        sxy += jnp.sum(xb * yb)
        sxx += jnp.sum(xb * xb)
        syy += jnp.sum(yb * yb)

    vx = jnp.sqrt(sxx / _N + 1e-8)
    vy = jnp.sqrt(syy / _N + 1e-8)
    rho = (sxy / _N) / (vx * vy + 1e-8)
    acc_ref[0] += (1.0 - rho) / _R

    @pl.when(r == _R - 1)
    def _():
        out_ref[0, 0] = acc_ref[0]


def kernel(preds, targets):
    p3 = preds.reshape(_R, 1, _N)
    t3 = targets.reshape(_R, 1, _N)
    out = pl.pallas_call(
        _body,
        grid=(_R,),
        in_specs=[
            pl.BlockSpec((1, 1, _N), lambda r: (r, 0, 0)),
            pl.BlockSpec((1, 1, _N), lambda r: (r, 0, 0)),
        ],
        out_specs=pl.BlockSpec(memory_space=pltpu.SMEM),
        out_shape=jax.ShapeDtypeStruct((1, 1), jnp.float32),
        scratch_shapes=[
            pltpu.SMEM((1,), jnp.float32),
            pltpu.VMEM((1, _N), jnp.float32),
            pltpu.VMEM((1, _N), jnp.float32),
        ],
    )(p3, t3)
    return out[0, 0]


# speedup vs baseline: 1.4663x; 1.3787x over previous
import jax
import jax.numpy as jnp
from jax.experimental import pallas as pl
from jax.experimental.pallas import tpu as pltpu

_TEMP_INV = 10.0
_N = 2048
_R = 8
_BI = 256
_NK = _N // _BI


def _body(p_ref, t_ref, out_ref, acc_ref, tp_ref, tt_ref):
    r = pl.program_id(0)

    @pl.when(r == 0)
    def _():
        acc_ref[0] = 0.0

    tp_ref[0, :] = jnp.zeros((_N,), jnp.float32)
    tt_ref[0, :] = jnp.zeros((_N,), jnp.float32)

    ap = p_ref[0, 0, :] * (0.5 * _TEMP_INV)
    at = t_ref[0, 0, :] * (0.5 * _TEMP_INV)

    sxy = 0.0
    sxx = 0.0
    syy = 0.0
    for i in range(_NK):
        lo = i * _BI
        hi = (i + 1) * _BI
        w = _N - lo

        pi = ap[lo:hi].reshape(_BI, 1)
        ps = ap[lo:].reshape(1, w)
        ti = at[lo:hi].reshape(_BI, 1)
        ts = at[lo:].reshape(1, w)

        bp_h = jnp.tanh((pi - ps).astype(jnp.bfloat16))
        bt_h = jnp.tanh((ti - ts).astype(jnp.bfloat16))
        ones_w = jnp.ones((1, w), jnp.bfloat16)
        ones_row = jnp.ones((1, _BI), jnp.bfloat16)
        dims = (((1,), (0,)), ((), ()))
        rs_p = jax.lax.dot_general(ones_w, bp_h.T, dims,
                                   preferred_element_type=jnp.float32)
        rs_t = jax.lax.dot_general(ones_w, bt_h.T, dims,
                                   preferred_element_type=jnp.float32)
        cs_p = jax.lax.dot_general(ones_row, bp_h, dims,
                                   preferred_element_type=jnp.float32)
        cs_t = jax.lax.dot_general(ones_row, bt_h, dims,
                                   preferred_element_type=jnp.float32)

        xb = 0.5 * (tp_ref[0, lo:hi] + rs_p[0, :])
        yb = 0.5 * (tt_ref[0, lo:hi] + rs_t[0, :])
        if i < _NK - 1:
            tp_ref[0, hi:] -= cs_p[0, _BI:]
            tt_ref[0, hi:] -= cs_t[0, _BI:]

        sxy += jnp.sum(xb * yb)
        sxx += jnp.sum(xb * xb)
        syy += jnp.sum(yb * yb)

    vx = jnp.sqrt(sxx / _N + 1e-8)
    vy = jnp.sqrt(syy / _N + 1e-8)
    rho = (sxy / _N) / (vx * vy + 1e-8)
    acc_ref[0] += (1.0 - rho) / _R

    @pl.when(r == _R - 1)
    def _():
        out_ref[0, 0] = acc_ref[0]


def kernel(preds, targets):
    p3 = preds.reshape(_R, 1, _N)
    t3 = targets.reshape(_R, 1, _N)
    out = pl.pallas_call(
        _body,
        grid=(_R,),
        in_specs=[
            pl.BlockSpec((1, 1, _N), lambda r: (r, 0, 0)),
            pl.BlockSpec((1, 1, _N), lambda r: (r, 0, 0)),
        ],
        out_specs=pl.BlockSpec(memory_space=pltpu.SMEM),
        out_shape=jax.ShapeDtypeStruct((1, 1), jnp.float32),
        scratch_shapes=[
            pltpu.SMEM((1,), jnp.float32),
            pltpu.VMEM((1, _N), jnp.float32),
            pltpu.VMEM((1, _N), jnp.float32),
        ],
    )(p3, t3)
    return out[0, 0]
